# Initial kernel scaffold; baseline (speedup 1.0000x reference)
import jax
import jax.numpy as jnp
from jax.experimental import pallas as pl


def kernel(Z, T, flux_table):
    raise NotImplementedError("write your pallas kernel here")


# speedup vs baseline: 4123.8388x; 4123.8388x over previous
import jax
import jax.numpy as jnp
from jax import lax
from jax.experimental import pallas as pl
from jax.experimental.pallas import tpu as pltpu
from jax.experimental.pallas import tpu_sc as plsc

NPTS = 100
TAB = NPTS * NPTS
NC, NS, L = 2, 16, 16
NW = NC * NS
CHUNK = 8000


def _body(z_hbm, t_hbm, tab_hbm, out_hbm, tab_v, z_v, t_v, o_v):
    nchunks = z_hbm.shape[0] // CHUNK
    wid = lax.axis_index("s") * NC + lax.axis_index("c")
    pltpu.sync_copy(tab_hbm, tab_v)

    @pl.loop(wid, nchunks, step=NW)
    def _chunk(k):
        off = k * CHUNK
        pltpu.sync_copy(z_hbm.at[pl.ds(off, CHUNK)], z_v)
        pltpu.sync_copy(t_hbm.at[pl.ds(off, CHUNK)], t_v)

        @pl.loop(0, CHUNK // L)
        def _vec(i):
            s = i * L
            t = t_v[pl.ds(s, L)]
            z = z_v[pl.ds(s, L)]
            tc = jnp.minimum(jnp.maximum((t - 0.1) * 10.0, 0.0), 98.0)
            zc = jnp.minimum(jnp.maximum((z - 0.01) * 100.0, 0.0), 98.0)
            it = tc.astype(jnp.int32)
            iz = zc.astype(jnp.int32)
            ft = tc - it.astype(jnp.float32)
            fz = zc - iz.astype(jnp.float32)
            base = it * NPTS + iz
            v00 = plsc.load_gather(tab_v, [base])
            v01 = plsc.load_gather(tab_v, [base + 1])
            v10 = plsc.load_gather(tab_v, [base + NPTS])
            v11 = plsc.load_gather(tab_v, [base + (NPTS + 1)])
            a = v00 + fz * (v01 - v00)
            b = v10 + fz * (v11 - v10)
            o_v[pl.ds(s, L)] = a + ft * (b - a)

        pltpu.sync_copy(o_v, out_hbm.at[pl.ds(off, CHUNK)])


def kernel(Z, T, flux_table):
    n = Z.shape[0]
    tab = flux_table.reshape(-1)
    mesh = plsc.VectorSubcoreMesh(core_axis_name="c", subcore_axis_name="s")
    f = pl.kernel(
        _body,
        out_type=jax.ShapeDtypeStruct((n,), jnp.float32),
        mesh=mesh,
        compiler_params=pltpu.CompilerParams(needs_layout_passes=False),
        scratch_types=[
            pltpu.VMEM((TAB,), jnp.float32),
            pltpu.VMEM((CHUNK,), jnp.float32),
            pltpu.VMEM((CHUNK,), jnp.float32),
            pltpu.VMEM((CHUNK,), jnp.float32),
        ],
    )
    return f(Z, T, tab)
